# baseline (speedup 1.0000x reference)
import functools

import jax
import jax.numpy as jnp
from jax import lax
from jax.experimental import pallas as pl
from jax.experimental.pallas import tpu as pltpu
from jax.experimental.pallas import tpu_sc as plsc

MAXP = 4096
DEMB = 64
NW = 32
LANES = 16


def kernel(input, embedding):
    S, W = input.shape
    k = min((S - 1) // 2, MAXP)
    rows_w = S // NW
    G = 16
    nt = rows_w // G

    mesh = plsc.VectorSubcoreMesh(core_axis_name="c", subcore_axis_name="s")

    @functools.partial(
        pl.kernel,
        mesh=mesh,
        out_type=jax.ShapeDtypeStruct((S, W, DEMB), jnp.float32),
        scratch_types=[
            pltpu.VMEM((rows_w, W), jnp.int32),
            pltpu.VMEM((2, G, W, DEMB), jnp.float32),
            pltpu.SemaphoreType.DMA,
            pltpu.SemaphoreType.DMA,
        ],
        compiler_params=pltpu.CompilerParams(use_tc_tiling_on_sc=False),
    )
    def body(inp_hbm, emb_hbm, out_hbm, idx_v, rows_v, gsem, wsem):
        nc = 2
        wid = lax.axis_index("s") * nc + lax.axis_index("c")
        s0 = wid * rows_w

        pltpu.sync_copy(inp_hbm.at[pl.ds(s0, rows_w)], idx_v)

        def transform(t):
            def fix(r, c):
                for q in range(W // LANES):
                    v = idx_v[r, pl.ds(q * LANES, LANES)]
                    v = jnp.clip(v, -MAXP, MAXP) + k
                    idx_v[r, pl.ds(q * LANES, LANES)] = v
                return c

            lax.fori_loop(t * G, (t + 1) * G, fix, 0)

        def fire_gathers(t, p):
            for a in range(G):
                pltpu.async_copy(
                    emb_hbm.at[idx_v.at[t * G + a]], rows_v.at[p, a], gsem
                )

        def drain_gathers():
            for a in range(G):
                pltpu.make_async_copy(
                    emb_hbm.at[idx_v.at[0]], rows_v.at[0, a], gsem
                ).wait()

        transform(0)
        fire_gathers(0, 0)

        def step(t, carry):
            p = lax.rem(t, 2)

            @pl.when(t + 1 < nt)
            def _():
                transform(t + 1)

            drain_gathers()

            @pl.when(t >= 1)
            def _():
                pltpu.make_async_copy(
                    rows_v.at[0], out_hbm.at[pl.ds(0, G)], wsem
                ).wait()

            @pl.when(t + 1 < nt)
            def _():
                fire_gathers(t + 1, 1 - p)

            pltpu.async_copy(rows_v.at[p], out_hbm.at[pl.ds(s0 + t * G, G)], wsem)
            return carry

        lax.fori_loop(0, nt, step, 0)
        pltpu.make_async_copy(rows_v.at[0], out_hbm.at[pl.ds(0, G)], wsem).wait()

    return body(input.astype(jnp.int32), embedding)
